# baseline (speedup 1.0000x reference)
import functools

import jax
import jax.numpy as jnp
from jax import lax
from jax.experimental import pallas as pl
from jax.experimental.pallas import tpu as pltpu
from jax.experimental.pallas import tpu_sc as plsc

NC = 2
NS = 16
L = 16
NW = NC * NS

H = 768
HV = H // L
EPS = 1e-12
CHUNK = 64


def _lane_sum(v):
    lanes = lax.iota(jnp.int32, L)
    dnums = lax.GatherDimensionNumbers(
        offset_dims=(), collapsed_slice_dims=(0,), start_index_map=(0,))
    for sh in (8, 4, 2, 1):
        v = v + lax.gather(v, (lanes ^ sh)[:, None], dnums, slice_sizes=(1,),
                           mode=lax.GatherScatterMode.PROMISE_IN_BOUNDS)
    return v


def _rsqrt(x):
    i = lax.bitcast_convert_type(x, jnp.int32)
    y = lax.bitcast_convert_type(jnp.int32(0x5F3759DF) - (i >> 1),
                                 jnp.float32)
    for _ in range(3):
        y = y * (1.5 - 0.5 * x * y * y)
    return y


def kernel(input_ids, position_ids, token_type_ids, word_embeddings,
           position_table, ln_gamma, ln_beta):
    B, S = input_ids.shape
    n_tok = B * S
    tok_per_w = n_tok // NW
    n_chunks = tok_per_w // CHUNK

    ids = input_ids.reshape(-1)
    pos = position_ids.reshape(-1)

    mesh = plsc.VectorSubcoreMesh(
        core_axis_name="c", subcore_axis_name="s",
        num_cores=NC, num_subcores=NS)

    @functools.partial(
        pl.kernel,
        out_type=jax.ShapeDtypeStruct((n_tok, H), jnp.float32),
        mesh=mesh,
        scratch_types=[
            pltpu.VMEM((CHUNK,), jnp.int32),
            pltpu.VMEM((CHUNK,), jnp.int32),
            pltpu.VMEM((CHUNK, H), jnp.float32),
            pltpu.VMEM((CHUNK, H), jnp.float32),
            pltpu.VMEM((H,), jnp.float32),
            pltpu.VMEM((H,), jnp.float32),
            pltpu.SemaphoreType.DMA,
            pltpu.SemaphoreType.DMA,
        ],
    )
    def run(ids_hbm, pos_hbm, wtab_hbm, ptab_hbm, gamma_hbm, beta_hbm,
            out_hbm, widx_v, pidx_v, wrows_v, prows_v, gamma_v, beta_v,
            wsem, psem):
        wid = lax.axis_index("s") * NC + lax.axis_index("c")
        base = wid * tok_per_w
        pltpu.sync_copy(gamma_hbm, gamma_v)
        pltpu.sync_copy(beta_hbm, beta_v)

        def chunk_body(c, _):
            cb = base + c * CHUNK
            pltpu.sync_copy(ids_hbm.at[pl.ds(cb, CHUNK)], widx_v)
            pltpu.sync_copy(pos_hbm.at[pl.ds(cb, CHUNK)], pidx_v)
            cw = pltpu.async_copy(wtab_hbm.at[widx_v], wrows_v, wsem)
            cp = pltpu.async_copy(ptab_hbm.at[pidx_v], prows_v, psem)
            cw.wait()
            cp.wait()

            def tok_body(t, _):
                acc = jnp.zeros((L,), jnp.float32)
                acc2 = jnp.zeros((L,), jnp.float32)
                for h in range(HV):
                    v = (wrows_v[t, pl.ds(h * L, L)]
                         + prows_v[t, pl.ds(h * L, L)])
                    wrows_v[t, pl.ds(h * L, L)] = v
                    acc = acc + v
                    acc2 = acc2 + v * v
                meanv = _lane_sum(acc) * (1.0 / H)
                varv = _lane_sum(acc2) * (1.0 / H) - meanv * meanv
                inv = _rsqrt(varv + EPS)
                for h in range(HV):
                    v = (wrows_v[t, pl.ds(h * L, L)] - meanv) * inv
                    wrows_v[t, pl.ds(h * L, L)] = (
                        v * gamma_v[pl.ds(h * L, L)]
                        + beta_v[pl.ds(h * L, L)])
                return 0

            lax.fori_loop(0, CHUNK, tok_body, 0)
            pltpu.sync_copy(wrows_v, out_hbm.at[pl.ds(cb, CHUNK)])
            return 0

        lax.fori_loop(0, n_chunks, chunk_body, 0)

    out = run(ids, pos, word_embeddings, position_table, ln_gamma, ln_beta)
    return out.reshape(B, S, H)
